# baseline (speedup 1.0000x reference)
import functools

import jax
import jax.numpy as jnp
from jax import lax
from jax.experimental import pallas as pl
from jax.experimental.pallas import tpu as pltpu
from jax.experimental.pallas import tpu_sc as plsc


def _bcast_body(tab_ref, out_ref):
    out_ref[...] = jnp.broadcast_to(tab_ref[...][None, :, :], out_ref.shape)


def kernel(inputs, position_embeddings):
    batch, seq_len, d_model = inputs.shape
    num_workers = 32
    rows_per_w = seq_len // num_workers
    chunk = 32
    n_chunks = rows_per_w // chunk
    mesh = plsc.VectorSubcoreMesh(core_axis_name="c", subcore_axis_name="s")

    @functools.partial(
        pl.kernel,
        mesh=mesh,
        out_type=jax.ShapeDtypeStruct((seq_len, d_model), jnp.float32),
        scratch_types=[
            pltpu.VMEM((chunk, d_model), jnp.float32),
            pltpu.VMEM((chunk, d_model), jnp.float32),
            pltpu.SemaphoreType.DMA,
            pltpu.SemaphoreType.DMA,
            pltpu.SemaphoreType.DMA,
            pltpu.SemaphoreType.DMA,
        ],
    )
    def sc_copy(table_hbm, out_hbm, buf0, buf1, rsem0, rsem1, wsem0, wsem1):
        wid = lax.axis_index("s") * 2 + lax.axis_index("c")
        base = wid * rows_per_w
        bufs = (buf0, buf1)
        rsems = (rsem0, rsem1)
        wsems = (wsem0, wsem1)

        def read(c):
            return pltpu.async_copy(
                table_hbm.at[pl.ds(base + c * chunk, chunk)], bufs[c % 2], rsems[c % 2]
            )

        reads = [read(c) if c < 2 else None for c in range(n_chunks)]
        writes = [None] * n_chunks
        for c in range(n_chunks):
            reads[c].wait()
            writes[c] = pltpu.async_copy(
                bufs[c % 2], out_hbm.at[pl.ds(base + c * chunk, chunk)], wsems[c % 2]
            )
            if c + 2 < n_chunks:
                writes[c].wait()
                reads[c + 2] = read(c + 2)
        for c in range(max(0, n_chunks - 2), n_chunks):
            writes[c].wait()

    batch0 = sc_copy(position_embeddings).reshape(1, seq_len, d_model)

    block_s = 1024
    rest = pl.pallas_call(
        _bcast_body,
        grid=(seq_len // block_s,),
        in_specs=[
            pl.BlockSpec((block_s, d_model), lambda i: (i, 0)),
        ],
        out_specs=pl.BlockSpec((batch - 1, block_s, d_model), lambda i: (0, i, 0)),
        out_shape=jax.ShapeDtypeStruct(
            (batch - 1, seq_len, d_model), position_embeddings.dtype
        ),
    )(position_embeddings)
    return jnp.concatenate([batch0, rest], axis=0)


# speedup vs baseline: 3.4861x; 3.4861x over previous
import jax
import jax.numpy as jnp
from jax.experimental import pallas as pl


def _bcast_body(tab_ref, out_ref):
    out_ref[...] = jnp.broadcast_to(tab_ref[...][None, :, :], out_ref.shape)


def kernel(inputs, position_embeddings):
    batch, seq_len, d_model = inputs.shape
    block_s = 1024
    grid = (seq_len // block_s,)
    out = pl.pallas_call(
        _bcast_body,
        grid=grid,
        in_specs=[
            pl.BlockSpec((block_s, d_model), lambda i: (i, 0)),
        ],
        out_specs=pl.BlockSpec((batch, block_s, d_model), lambda i: (0, i, 0)),
        out_shape=jax.ShapeDtypeStruct((batch, seq_len, d_model), position_embeddings.dtype),
    )(position_embeddings)
    return out
